# Initial kernel scaffold; baseline (speedup 1.0000x reference)
import jax
import jax.numpy as jnp
from jax.experimental import pallas as pl


def kernel(z, x, edge_index, We1, be1, We2, be2, Wf1, bf1, Wf2, bf2, Wo, bo):
    raise NotImplementedError("write your pallas kernel here")


# speedup vs baseline: 11.7363x; 11.7363x over previous
import functools

import jax
import jax.numpy as jnp
from jax import lax
from jax.experimental import pallas as pl
from jax.experimental.pallas import tpu as pltpu
from jax.experimental.pallas import tpu_sc as plsc

N = 10000
NP = 10240
E = 320000
D = 128
CH = 128
NCH = E // CH
BR = 512
GRID = NP // BR
RPT = NP // 16
SLOPE = (1.0 / 8.0 + 1.0 / 3.0) / 2.0


def _mesh():
    return plsc.VectorSubcoreMesh(core_axis_name="c", subcore_axis_name="s")


def _sc_deg_body(dst_hbm, out_hbm, didx, ones_v, zbuf, acc):
    cid = lax.axis_index("c")
    sid = lax.axis_index("s")
    wid = cid * 16 + sid

    for l in range(8):
        ones_v[pl.ds(l * 16, 16)] = jnp.full((16,), 1.0, jnp.float32)
    zeros16 = jnp.zeros((16,), jnp.float32)

    @pl.loop(0, RPT // 16)
    def _zero(i):
        zbuf[pl.ds(i * 16, 16)] = zeros16

    pltpu.sync_copy(zbuf, acc.at[pl.ds(sid * RPT, RPT)])
    plsc.subcore_barrier()

    nj = jnp.where(wid < NCH - 32 * (NCH // 32), NCH // 32 + 1, NCH // 32)

    @pl.loop(0, nj)
    def _edges(j):
        off = (wid + 32 * j) * CH
        pltpu.sync_copy(dst_hbm.at[pl.ds(off, CH)], didx)
        pltpu.sync_copy(ones_v, acc.at[didx], add=True)

    plsc.subcore_barrier()
    pltpu.sync_copy(acc.at[pl.ds(sid * RPT, RPT)],
                    out_hbm.at[cid, pl.ds(sid * RPT, RPT)])


def _sc_deg(dst):
    f = functools.partial(
        pl.kernel,
        out_type=jax.ShapeDtypeStruct((2, NP), jnp.float32),
        mesh=_mesh(),
        scratch_types=[
            pltpu.VMEM((CH,), jnp.int32),
            pltpu.VMEM((CH,), jnp.float32),
            pltpu.VMEM((RPT,), jnp.float32),
            pltpu.VMEM_SHARED((NP,), jnp.float32),
        ],
    )(_sc_deg_body)
    return f(dst)


def _sc_agg_body(g_hbm, src_hbm, dst_hbm, out_hbm, sidx, didx, rows, acc, sem):
    cid = lax.axis_index("c")
    sid = lax.axis_index("s")

    zeros16 = jnp.zeros((16,), jnp.float32)

    @pl.loop(0, CH)
    def _zrow(r):
        for l in range(D // 16):
            rows[r, pl.ds(l * 16, 16)] = zeros16

    for k in range(RPT // CH):
        pltpu.sync_copy(rows, acc.at[pl.ds(sid * RPT + k * CH, CH)])
    plsc.subcore_barrier()

    nj = jnp.where(sid < NCH - 16 * (NCH // 16), NCH // 16 + 1, NCH // 16)

    @pl.loop(0, nj)
    def _edges(j):
        off = (sid + 16 * j) * CH
        pltpu.sync_copy(src_hbm.at[pl.ds(off, CH)], sidx)
        pltpu.sync_copy(dst_hbm.at[pl.ds(off, CH)], didx)
        pltpu.async_copy(g_hbm.at[cid].at[sidx], rows, sem).wait()
        pltpu.sync_copy(rows, acc.at[didx], add=True)

    plsc.subcore_barrier()
    for k in range(RPT // CH):
        pltpu.sync_copy(acc.at[pl.ds(sid * RPT + k * CH, CH)],
                        out_hbm.at[cid].at[pl.ds(sid * RPT + k * CH, CH)])


def _sc_agg(g, src, dst):
    f = functools.partial(
        pl.kernel,
        out_type=jax.ShapeDtypeStruct((2, NP, D), jnp.float32),
        mesh=_mesh(),
        scratch_types=[
            pltpu.VMEM((CH,), jnp.int32),
            pltpu.VMEM((CH,), jnp.int32),
            pltpu.VMEM((CH, D), jnp.float32),
            pltpu.VMEM_SHARED((NP, D), jnp.float32),
            pltpu.SemaphoreType.DMA,
        ],
    )(_sc_agg_body)
    return f(g, src, dst)


def _tc1_body(z_ref, x_ref, d2_ref, w_ref, g_ref, dinv_ref):
    deg = d2_ref[0] + d2_ref[1] + 1.0
    dinv = lax.rsqrt(deg)
    dinv_ref[...] = dinv
    g_ref[0] = dinv * jnp.dot(z_ref[...], w_ref[0],
                              preferred_element_type=jnp.float32)
    g_ref[1] = dinv * jnp.dot(x_ref[...], w_ref[1],
                              preferred_element_type=jnp.float32)


def _tc1(z_pad, x_pad, deg2, w1):
    return pl.pallas_call(
        _tc1_body,
        grid=(GRID,),
        in_specs=[
            pl.BlockSpec((BR, D), lambda i: (i, 0)),
            pl.BlockSpec((BR, D), lambda i: (i, 0)),
            pl.BlockSpec((2, BR, 1), lambda i: (0, i, 0)),
            pl.BlockSpec((2, D, D), lambda i: (0, 0, 0)),
        ],
        out_specs=[
            pl.BlockSpec((2, BR, D), lambda i: (0, i, 0)),
            pl.BlockSpec((BR, 1), lambda i: (i, 0)),
        ],
        out_shape=[
            jax.ShapeDtypeStruct((2, NP, D), jnp.float32),
            jax.ShapeDtypeStruct((NP, 1), jnp.float32),
        ],
    )(z_pad, x_pad, deg2, w1)


def _tc2_body(agg_ref, g_ref, dinv_ref, b_ref, w_ref, out_ref):
    dinv = dinv_ref[...]
    for c in range(2):
        u = dinv * (agg_ref[c] + g_ref[c]) + b_ref[c]
        u = jnp.where(u >= 0, u, u * SLOPE)
        out_ref[c] = dinv * jnp.dot(u, w_ref[c],
                                    preferred_element_type=jnp.float32)


def _tc2(agg1, g1, dinv, b1, w2):
    return pl.pallas_call(
        _tc2_body,
        grid=(GRID,),
        in_specs=[
            pl.BlockSpec((2, BR, D), lambda i: (0, i, 0)),
            pl.BlockSpec((2, BR, D), lambda i: (0, i, 0)),
            pl.BlockSpec((BR, 1), lambda i: (i, 0)),
            pl.BlockSpec((2, D), lambda i: (0, 0)),
            pl.BlockSpec((2, D, D), lambda i: (0, 0, 0)),
        ],
        out_specs=pl.BlockSpec((2, BR, D), lambda i: (0, i, 0)),
        out_shape=jax.ShapeDtypeStruct((2, NP, D), jnp.float32),
    )(agg1, g1, dinv, b1, w2)


def _tc3_body(agg_ref, g_ref, dinv_ref, b_ref, wo_ref, bo_ref, out_ref):
    dinv = dinv_ref[...]
    zz = jnp.tanh(dinv * (agg_ref[0] + g_ref[0]) + b_ref[0])
    xx = jnp.tanh(dinv * (agg_ref[1] + g_ref[1]) + b_ref[1])
    out_ref[...] = jnp.dot(zz * xx, wo_ref[...],
                           preferred_element_type=jnp.float32) + bo_ref[...]


def _tc3(agg2, g2, dinv, b2, Wo, bo):
    return pl.pallas_call(
        _tc3_body,
        grid=(GRID,),
        in_specs=[
            pl.BlockSpec((2, BR, D), lambda i: (0, i, 0)),
            pl.BlockSpec((2, BR, D), lambda i: (0, i, 0)),
            pl.BlockSpec((BR, 1), lambda i: (i, 0)),
            pl.BlockSpec((2, D), lambda i: (0, 0)),
            pl.BlockSpec((D, 1), lambda i: (0, 0)),
            pl.BlockSpec((1,), lambda i: (0,)),
        ],
        out_specs=pl.BlockSpec((BR, 1), lambda i: (i, 0)),
        out_shape=jax.ShapeDtypeStruct((NP, 1), jnp.float32),
    )(agg2, g2, dinv, b2, Wo, bo)


@jax.jit
def kernel(z, x, edge_index, We1, be1, We2, be2, Wf1, bf1, Wf2, bf2, Wo, bo):
    src = edge_index[0]
    dst = edge_index[1]

    z_pad = jnp.pad(z, ((0, NP - N), (0, 0)))
    x_pad = jnp.pad(x, ((0, NP - N), (0, 0)))
    w1 = jnp.stack([We1, Wf1])
    w2 = jnp.stack([We2, Wf2])
    b1 = jnp.stack([be1, bf1])
    b2 = jnp.stack([be2, bf2])

    deg2 = _sc_deg(dst)
    deg2 = deg2[:, :, None]

    g1, dinv = _tc1(z_pad, x_pad, deg2, w1)
    agg1 = _sc_agg(g1, src, dst)
    g2 = _tc2(agg1, g1, dinv, b1, w2)
    agg2 = _sc_agg(g2, src, dst)
    out = _tc3(agg2, g2, dinv, b2, Wo, bo)
    return out[:N]


# speedup vs baseline: 18.0741x; 1.5400x over previous
import functools

import jax
import jax.numpy as jnp
from jax import lax
from jax.experimental import pallas as pl
from jax.experimental.pallas import tpu as pltpu
from jax.experimental.pallas import tpu_sc as plsc

N = 10000
NP = 10240
E = 320000
D = 128
CH = 128
NCH = E // CH
BR = 512
GRID = NP // BR
RPT = NP // 16
SLOPE = (1.0 / 8.0 + 1.0 / 3.0) / 2.0


def _mesh():
    return plsc.VectorSubcoreMesh(core_axis_name="c", subcore_axis_name="s")


def _sc_deg_body(dst_hbm, out_hbm, didx, ones_v, zbuf, acc):
    cid = lax.axis_index("c")
    sid = lax.axis_index("s")
    wid = cid * 16 + sid

    for l in range(8):
        ones_v[pl.ds(l * 16, 16)] = jnp.full((16,), 1.0, jnp.float32)
    zeros16 = jnp.zeros((16,), jnp.float32)

    @pl.loop(0, RPT // 16)
    def _zero(i):
        zbuf[pl.ds(i * 16, 16)] = zeros16

    pltpu.sync_copy(zbuf, acc.at[pl.ds(sid * RPT, RPT)])
    plsc.subcore_barrier()

    nj = jnp.where(wid < NCH - 32 * (NCH // 32), NCH // 32 + 1, NCH // 32)

    @pl.loop(0, nj)
    def _edges(j):
        off = (wid + 32 * j) * CH
        pltpu.sync_copy(dst_hbm.at[pl.ds(off, CH)], didx)
        pltpu.sync_copy(ones_v, acc.at[didx], add=True)

    plsc.subcore_barrier()
    pltpu.sync_copy(acc.at[pl.ds(sid * RPT, RPT)],
                    out_hbm.at[cid, pl.ds(sid * RPT, RPT)])


def _sc_deg(dst):
    f = functools.partial(
        pl.kernel,
        out_type=jax.ShapeDtypeStruct((2, NP), jnp.float32),
        mesh=_mesh(),
        scratch_types=[
            pltpu.VMEM((CH,), jnp.int32),
            pltpu.VMEM((CH,), jnp.float32),
            pltpu.VMEM((RPT,), jnp.float32),
            pltpu.VMEM_SHARED((NP,), jnp.float32),
        ],
    )(_sc_deg_body)
    return f(dst)


NJ = NCH // 16
NREM = NCH - 16 * NJ


def _sc_agg_body(g_hbm, src_hbm, dst_hbm, out_hbm,
                 sidx0, didx0, rows0, sidx1, didx1, rows1,
                 acc, sem0, sem1):
    cid = lax.axis_index("c")
    sid = lax.axis_index("s")
    gv = g_hbm.at[cid]

    zeros16 = jnp.zeros((16,), jnp.float32)

    @pl.loop(0, CH)
    def _zrow(r):
        for l in range(D // 16):
            rows0[r, pl.ds(l * 16, 16)] = zeros16

    for k in range(RPT // CH):
        pltpu.sync_copy(rows0, acc.at[pl.ds(sid * RPT + k * CH, CH)])
    plsc.subcore_barrier()

    sbuf = (sidx0, sidx1)
    dbuf = (didx0, didx1)
    rbuf = (rows0, rows1)
    sems = (sem0, sem1)

    def load_idx(j, b):
        off = (sid + 16 * j) * CH
        pltpu.sync_copy(src_hbm.at[pl.ds(off, CH)], sbuf[b])
        pltpu.sync_copy(dst_hbm.at[pl.ds(off, CH)], dbuf[b])

    def start_gather(b):
        pltpu.async_copy(gv.at[sbuf[b]], rbuf[b], sems[b])

    def finish(b):
        pltpu.make_async_copy(gv.at[sbuf[b]], rbuf[b], sems[b]).wait()
        pltpu.sync_copy(rbuf[b], acc.at[dbuf[b]], add=True)

    load_idx(0, 0)
    start_gather(0)

    @pl.loop(0, NJ, step=2)
    def _edges(j):
        load_idx(j + 1, 1)
        start_gather(1)
        finish(0)

        @pl.when(j + 2 < NJ)
        def _():
            load_idx(j + 2, 0)
            start_gather(0)

        finish(1)

    @pl.when(sid < NREM)
    def _rem():
        off = (16 * NJ + sid) * CH
        pltpu.sync_copy(src_hbm.at[pl.ds(off, CH)], sidx0)
        pltpu.sync_copy(dst_hbm.at[pl.ds(off, CH)], didx0)
        pltpu.async_copy(gv.at[sidx0], rows0, sem0).wait()
        pltpu.sync_copy(rows0, acc.at[didx0], add=True)

    plsc.subcore_barrier()
    for k in range(RPT // CH):
        pltpu.sync_copy(acc.at[pl.ds(sid * RPT + k * CH, CH)],
                        out_hbm.at[cid].at[pl.ds(sid * RPT + k * CH, CH)])


def _sc_agg(g, src, dst):
    f = functools.partial(
        pl.kernel,
        out_type=jax.ShapeDtypeStruct((2, NP, D), jnp.float32),
        mesh=_mesh(),
        scratch_types=[
            pltpu.VMEM((CH,), jnp.int32),
            pltpu.VMEM((CH,), jnp.int32),
            pltpu.VMEM((CH, D), jnp.float32),
            pltpu.VMEM((CH,), jnp.int32),
            pltpu.VMEM((CH,), jnp.int32),
            pltpu.VMEM((CH, D), jnp.float32),
            pltpu.VMEM_SHARED((NP, D), jnp.float32),
            pltpu.SemaphoreType.DMA,
            pltpu.SemaphoreType.DMA,
        ],
    )(_sc_agg_body)
    return f(g, src, dst)


def _tc1_body(z_ref, x_ref, d2_ref, w_ref, g_ref, dinv_ref):
    deg = d2_ref[0] + d2_ref[1] + 1.0
    dinv = lax.rsqrt(deg)
    dinv_ref[...] = dinv
    g_ref[0] = dinv * jnp.dot(z_ref[...], w_ref[0],
                              preferred_element_type=jnp.float32)
    g_ref[1] = dinv * jnp.dot(x_ref[...], w_ref[1],
                              preferred_element_type=jnp.float32)


def _tc1(z_pad, x_pad, deg2, w1):
    return pl.pallas_call(
        _tc1_body,
        grid=(GRID,),
        in_specs=[
            pl.BlockSpec((BR, D), lambda i: (i, 0)),
            pl.BlockSpec((BR, D), lambda i: (i, 0)),
            pl.BlockSpec((2, BR, 1), lambda i: (0, i, 0)),
            pl.BlockSpec((2, D, D), lambda i: (0, 0, 0)),
        ],
        out_specs=[
            pl.BlockSpec((2, BR, D), lambda i: (0, i, 0)),
            pl.BlockSpec((BR, 1), lambda i: (i, 0)),
        ],
        out_shape=[
            jax.ShapeDtypeStruct((2, NP, D), jnp.float32),
            jax.ShapeDtypeStruct((NP, 1), jnp.float32),
        ],
    )(z_pad, x_pad, deg2, w1)


def _tc2_body(agg_ref, g_ref, dinv_ref, b_ref, w_ref, out_ref):
    dinv = dinv_ref[...]
    for c in range(2):
        u = dinv * (agg_ref[c] + g_ref[c]) + b_ref[c]
        u = jnp.where(u >= 0, u, u * SLOPE)
        out_ref[c] = dinv * jnp.dot(u, w_ref[c],
                                    preferred_element_type=jnp.float32)


def _tc2(agg1, g1, dinv, b1, w2):
    return pl.pallas_call(
        _tc2_body,
        grid=(GRID,),
        in_specs=[
            pl.BlockSpec((2, BR, D), lambda i: (0, i, 0)),
            pl.BlockSpec((2, BR, D), lambda i: (0, i, 0)),
            pl.BlockSpec((BR, 1), lambda i: (i, 0)),
            pl.BlockSpec((2, D), lambda i: (0, 0)),
            pl.BlockSpec((2, D, D), lambda i: (0, 0, 0)),
        ],
        out_specs=pl.BlockSpec((2, BR, D), lambda i: (0, i, 0)),
        out_shape=jax.ShapeDtypeStruct((2, NP, D), jnp.float32),
    )(agg1, g1, dinv, b1, w2)


def _tc3_body(agg_ref, g_ref, dinv_ref, b_ref, wo_ref, bo_ref, out_ref):
    dinv = dinv_ref[...]
    zz = jnp.tanh(dinv * (agg_ref[0] + g_ref[0]) + b_ref[0])
    xx = jnp.tanh(dinv * (agg_ref[1] + g_ref[1]) + b_ref[1])
    out_ref[...] = jnp.dot(zz * xx, wo_ref[...],
                           preferred_element_type=jnp.float32) + bo_ref[...]


def _tc3(agg2, g2, dinv, b2, Wo, bo):
    return pl.pallas_call(
        _tc3_body,
        grid=(GRID,),
        in_specs=[
            pl.BlockSpec((2, BR, D), lambda i: (0, i, 0)),
            pl.BlockSpec((2, BR, D), lambda i: (0, i, 0)),
            pl.BlockSpec((BR, 1), lambda i: (i, 0)),
            pl.BlockSpec((2, D), lambda i: (0, 0)),
            pl.BlockSpec((D, 1), lambda i: (0, 0)),
            pl.BlockSpec((1,), lambda i: (0,)),
        ],
        out_specs=pl.BlockSpec((BR, 1), lambda i: (i, 0)),
        out_shape=jax.ShapeDtypeStruct((NP, 1), jnp.float32),
    )(agg2, g2, dinv, b2, Wo, bo)


@jax.jit
def kernel(z, x, edge_index, We1, be1, We2, be2, Wf1, bf1, Wf2, bf2, Wo, bo):
    src = edge_index[0]
    dst = edge_index[1]

    z_pad = jnp.pad(z, ((0, NP - N), (0, 0)))
    x_pad = jnp.pad(x, ((0, NP - N), (0, 0)))
    w1 = jnp.stack([We1, Wf1])
    w2 = jnp.stack([We2, Wf2])
    b1 = jnp.stack([be1, bf1])
    b2 = jnp.stack([be2, bf2])

    deg2 = _sc_deg(dst)
    deg2 = deg2[:, :, None]

    g1, dinv = _tc1(z_pad, x_pad, deg2, w1)
    agg1 = _sc_agg(g1, src, dst)
    g2 = _tc2(agg1, g1, dinv, b1, w2)
    agg2 = _sc_agg(g2, src, dst)
    out = _tc3(agg2, g2, dinv, b2, Wo, bo)
    return out[:N]
